# baseline (speedup 1.0000x reference)
import functools

import jax
import jax.numpy as jnp
from jax import lax
from jax.experimental import pallas as pl
from jax.experimental.pallas import tpu as pltpu
from jax.experimental.pallas import tpu_sc as plsc

B, S, E = 4, 2048, 768
NC, NS = 2, 16
NW = NC * NS
S_PER_W = S // NW
CH = 8
CHW = CH * E
N_CHUNKS = S_PER_W // CH
NBUF = 3
LANES = 16


def _sc_body(x_hbm, tab_hbm, out_hbm, t_v, x_v, in_sem, out_sem):
    wid = lax.axis_index("s") * NC + lax.axis_index("c")
    w0 = wid * S_PER_W * E

    def in_copies(c, slot):
        base = w0 + c * CHW
        return [
            pltpu.make_async_copy(
                tab_hbm.at[pl.ds(base, CHW)], t_v.at[slot], in_sem.at[slot]),
            pltpu.make_async_copy(
                x_hbm.at[:, pl.ds(base, CHW)], x_v.at[slot], in_sem.at[slot]),
        ]

    def out_copies(c, slot):
        base = w0 + c * CHW
        return [pltpu.make_async_copy(
            x_v.at[slot], out_hbm.at[:, pl.ds(base, CHW)], out_sem.at[slot])]

    for cp in in_copies(0, 0):
        cp.start()

    def chunk_body(c, _):
        slot = lax.rem(c, NBUF)

        @pl.when(c + 1 < N_CHUNKS)
        def _prefetch():
            nslot = lax.rem(c + 1, NBUF)

            @pl.when(c >= 2)
            def _drain_prev_out():
                for cp in out_copies(c - 2, nslot):
                    cp.wait()

            for cp in in_copies(c + 1, nslot):
                cp.start()

        for cp in in_copies(c, slot):
            cp.wait()

        @plsc.parallel_loop(0, CHW, step=LANES, unroll=8)
        def _add(i):
            t = t_v[slot, pl.ds(i, LANES)]
            for b in range(B):
                plsc.addupdate(x_v.at[slot, b, pl.ds(i, LANES)], t)

        for cp in out_copies(c, slot):
            cp.start()
        return 0

    lax.fori_loop(0, N_CHUNKS, chunk_body, 0)

    for c in (N_CHUNKS - 3, N_CHUNKS - 2, N_CHUNKS - 1):
        for cp in out_copies(c, c % NBUF):
            cp.wait()


_sc_call = functools.partial(
    pl.kernel,
    out_type=jax.ShapeDtypeStruct((B, S * E), jnp.float32),
    mesh=plsc.VectorSubcoreMesh(core_axis_name="c", subcore_axis_name="s"),
    scratch_types=[
        pltpu.VMEM((NBUF, CHW), jnp.float32),
        pltpu.VMEM((NBUF, B, CHW), jnp.float32),
        pltpu.SemaphoreType.DMA((NBUF,)),
        pltpu.SemaphoreType.DMA((NBUF,)),
    ],
)(_sc_body)


def kernel(x, pos_table):
    b, s, e = x.shape
    out = _sc_call(x.reshape(b, s * e), pos_table.reshape(s * e))
    return out.reshape(b, s, e)
